# baseline (speedup 1.0000x reference)
import jax
import jax.numpy as jnp
from jax.experimental import pallas as pl

SHAPE = (16384, 128)
T_USED = 10
ROW_BLOCK = 1024


def _body(ap_ref, h_ref, o_ref):
    t = pl.program_id(1)

    @pl.when(t == 0)
    def _():
        o_ref[...] = ap_ref[...] + h_ref[0]

    @pl.when(t != 0)
    def _():
        o_ref[...] += h_ref[0]


def kernel(action_potential, action_potential_history):
    n_rows = SHAPE[0] // ROW_BLOCK
    return pl.pallas_call(
        _body,
        grid=(n_rows, T_USED),
        in_specs=[
            pl.BlockSpec((ROW_BLOCK, SHAPE[1]), lambda r, t: (r, 0)),
            pl.BlockSpec((1, ROW_BLOCK, SHAPE[1]), lambda r, t: (t, r, 0)),
        ],
        out_specs=pl.BlockSpec((ROW_BLOCK, SHAPE[1]), lambda r, t: (r, 0)),
        out_shape=jax.ShapeDtypeStruct(SHAPE, jnp.float32),
    )(action_potential, action_potential_history)


# speedup vs baseline: 3.2241x; 3.2241x over previous
import jax
import jax.numpy as jnp
from jax.experimental import pallas as pl

SHAPE = (16384, 128)
T_USED = 10
ROW_BLOCK = 1024


def _body(ap_ref, h_ref, o_ref):
    acc = ap_ref[...]
    for t in range(T_USED):
        acc = acc + h_ref[t]
    o_ref[...] = acc


def kernel(action_potential, action_potential_history):
    n_rows = SHAPE[0] // ROW_BLOCK
    return pl.pallas_call(
        _body,
        grid=(n_rows,),
        in_specs=[
            pl.BlockSpec((ROW_BLOCK, SHAPE[1]), lambda r: (r, 0)),
            pl.BlockSpec((T_USED, ROW_BLOCK, SHAPE[1]), lambda r: (0, r, 0)),
        ],
        out_specs=pl.BlockSpec((ROW_BLOCK, SHAPE[1]), lambda r: (r, 0)),
        out_shape=jax.ShapeDtypeStruct(SHAPE, jnp.float32),
    )(action_potential, action_potential_history)
